# baseline (speedup 1.0000x reference)
import functools

import jax
import jax.numpy as jnp
from jax import lax
from jax.experimental import pallas as pl
from jax.experimental.pallas import tpu as pltpu
from jax.experimental.pallas import tpu_sc as plsc

NC = 2
NS = 16
NW = NC * NS

CHUNK = 128
NBUF = 4


def _gather_body(n_l, n_jj, ids_hbm, table_hbm, out_hbm, idsv, idx_v, rows_v,
                 trans_v, in_sems, out_sems):
  wid = lax.axis_index("s") * NC + lax.axis_index("c")
  bw = n_jj * CHUNK
  n_chunks = n_l * n_jj
  lane = lax.iota(jnp.int32, 16)

  pltpu.sync_copy(ids_hbm.at[:, pl.ds(wid * bw, bw)], idsv)

  def idx_body(g, carry):
    l = g // n_jj
    jj = g % n_jj
    for k in range(CHUNK // 16):
      src = plsc.load_gather(
          idsv, [jnp.broadcast_to(l, (16,)), lane + (jj * CHUNK + 16 * k)])
      idx_v[g, pl.ds(16 * k, 16)] = src
    return carry

  lax.fori_loop(0, n_chunks, idx_body, 0)

  def start_gather(g, b):
    pltpu.async_copy(table_hbm.at[idx_v.at[g]], rows_v.at[b],
                     in_sems.at[b])

  for b in range(NBUF):
    start_gather(b, b)

  def round_body(r, carry):
    for b in range(NBUF):
      g = r * NBUF + b
      pltpu.make_async_copy(table_hbm.at[idx_v.at[g]], rows_v.at[b],
                            in_sems.at[b]).wait()

      @pl.when(g >= NBUF)
      def _wait_out():
        pltpu.make_async_copy(trans_v.at[b], out_hbm.at[0, :, 0],
                              out_sems.at[b]).wait()

      def i_body(i, icarry):
        for ih in range(8):
          h = i * 8 + ih
          hvec = jnp.broadcast_to(h, (16,))
          for k in range(CHUNK // 16):
            src = plsc.load_gather(rows_v.at[b], [lane + 16 * k, hvec])
            trans_v[b, i, pl.ds(ih * CHUNK + 16 * k, 16)] = src
        return icarry

      lax.fori_loop(0, 8, i_body, 0)

      l = g // n_jj
      j = wid * n_jj + g % n_jj
      pltpu.async_copy(trans_v.at[b], out_hbm.at[l, :, j], out_sems.at[b])

      @pl.when(g + NBUF < n_chunks)
      def _refill():
        start_gather(g + NBUF, b)

    return carry

  lax.fori_loop(0, n_chunks // NBUF, round_body, 0)

  for b in range(NBUF):
    pltpu.make_async_copy(trans_v.at[b], out_hbm.at[0, :, 0],
                          out_sems.at[b]).wait()


def _impl(input_ids, latents, W_emb):
  del latents
  nb, nl = input_ids.shape
  hidden = W_emb.shape[1]
  n_j = nb // CHUNK
  n_jj = n_j // NW
  ids_t = input_ids.T.astype(jnp.int32)

  mesh = plsc.VectorSubcoreMesh(core_axis_name="c", subcore_axis_name="s",
                                num_cores=NC, num_subcores=NS)
  fn = pl.kernel(
      functools.partial(_gather_body, nl, n_jj),
      out_type=jax.ShapeDtypeStruct((nl, hidden // 8, n_j, 8 * CHUNK),
                                    jnp.float32),
      mesh=mesh,
      scratch_types=[
          pltpu.VMEM((nl, n_jj * CHUNK), jnp.int32),
          pltpu.VMEM((nl * n_jj, CHUNK), jnp.int32),
          pltpu.VMEM((NBUF, CHUNK, hidden), jnp.float32),
          pltpu.VMEM((NBUF, hidden // 8, 8 * CHUNK), jnp.float32),
          pltpu.SemaphoreType.DMA((NBUF,)),
          pltpu.SemaphoreType.DMA((NBUF,)),
      ],
      compiler_params=pltpu.CompilerParams(use_tc_tiling_on_sc=False,
                                           needs_layout_passes=False),
  )
  x4 = fn(ids_t, W_emb)
  x5 = x4.reshape(nl, hidden // 8, n_j, 8, CHUNK)
  return x5.transpose(2, 4, 0, 1, 3).reshape(nb, nl, hidden)


kernel = jax.jit(_impl)


# speedup vs baseline: 1.2504x; 1.2504x over previous
import functools

import jax
import jax.numpy as jnp
from jax import lax
from jax.experimental import pallas as pl
from jax.experimental.pallas import tpu as pltpu
from jax.experimental.pallas import tpu_sc as plsc

NC = 2
NS = 16
NW = NC * NS

CHUNK = 80
NBUF = 4


def _gather_body(n_chunks, n_l, ids_hbm, table_hbm, out_hbm, idx_v, rows_v,
                 in_sems, out_sems):
  wid = lax.axis_index("s") * NC + lax.axis_index("c")
  chunk0 = wid * n_chunks
  hidden = table_hbm.shape[1]
  rpc = CHUNK // n_l

  pltpu.sync_copy(ids_hbm.at[pl.ds(chunk0, n_chunks)], idx_v)

  def start_gather(g, b):
    pltpu.async_copy(table_hbm.at[idx_v.at[g]], rows_v.at[b],
                     in_sems.at[b])

  for b in range(NBUF):
    start_gather(b, b)

  def round_body(r, carry):
    for b in range(NBUF):
      g = r * NBUF + b
      pltpu.make_async_copy(table_hbm.at[idx_v.at[g]], rows_v.at[b],
                            in_sems.at[b]).wait()
      b0 = (chunk0 + g) * rpc
      for q in range(rpc):
        pltpu.async_copy(rows_v.at[b, pl.ds(q * n_l, n_l)],
                         out_hbm.at[b0 + q], out_sems.at[b])
      for q in range(rpc):
        pltpu.make_async_copy(rows_v.at[b, pl.ds(q * n_l, n_l)],
                              out_hbm.at[b0 + q], out_sems.at[b]).wait()

      @pl.when(g + NBUF < n_chunks)
      def _refill():
        start_gather(g + NBUF, b)

    return carry

  lax.fori_loop(0, n_chunks // NBUF, round_body, 0)


def _impl(input_ids, latents, W_emb):
  del latents
  nb, nl = input_ids.shape
  hidden = W_emb.shape[1]
  n_flat = nb * nl
  n_chunks = n_flat // (NW * CHUNK)
  ids2d = input_ids.reshape(-1, CHUNK).astype(jnp.int32)

  mesh = plsc.VectorSubcoreMesh(core_axis_name="c", subcore_axis_name="s",
                                num_cores=NC, num_subcores=NS)
  fn = pl.kernel(
      functools.partial(_gather_body, n_chunks, nl),
      out_type=jax.ShapeDtypeStruct((nb, nl, hidden), jnp.float32),
      mesh=mesh,
      scratch_types=[
          pltpu.VMEM((n_chunks, CHUNK), jnp.int32),
          pltpu.VMEM((NBUF, CHUNK, hidden), jnp.float32),
          pltpu.SemaphoreType.DMA((NBUF,)),
          pltpu.SemaphoreType.DMA((NBUF,)),
      ],
      compiler_params=pltpu.CompilerParams(use_tc_tiling_on_sc=False),
  )
  return fn(ids2d, W_emb)


kernel = jax.jit(_impl)
